# baseline (speedup 1.0000x reference)
import functools

import jax
import jax.numpy as jnp
from jax import lax
from jax.experimental import pallas as pl
from jax.experimental.pallas import tpu as pltpu
from jax.experimental.pallas import tpu_sc as plsc

_N = 10000
_E = 160000
_R = 21
_H = 256
_L = 3
_DH = 128

_NPAD = 10240
_R1 = _R + 1
_HALF = 128
_NC = 2
_NS = 16
_EB = 128
_EP = 163840
_CHUNK = _EP // _NS
_NBLK = _CHUNK // _EB
_ROWS_PER_SUB = _NPAD // _NS


def _sc_scatter_body(z2, ridx, wvec, dstv, out,
                     idx0, idx1, dst0, dst1, w0, w1,
                     rows0, rows1, acc, g0, g1, m0, m1):
    c = lax.axis_index("c")
    s = lax.axis_index("s")

    idxb = (idx0, idx1)
    dstb = (dst0, dst1)
    wb = (w0, w1)
    rowsb = (rows0, rows1)
    gsem = (g0, g1)
    msem = (m0, m1)

    def _meta_start(jj, p):
        pltpu.async_copy(ridx.at[c, s, jj], idxb[p], msem[p])
        pltpu.async_copy(dstv.at[s, jj], dstb[p], msem[p])
        pltpu.async_copy(wvec.at[s, jj], wb[p], msem[p])

    def _meta_wait(p):
        pltpu.make_async_copy(ridx.at[c, s, 0], idxb[p], msem[p]).wait()
        pltpu.make_async_copy(dstv.at[s, 0], dstb[p], msem[p]).wait()
        pltpu.make_async_copy(wvec.at[s, 0], wb[p], msem[p]).wait()

    def _gather_start(p):
        pltpu.async_copy(z2.at[idxb[p]], rowsb[p], gsem[p])

    def _gather_wait(p):
        pltpu.make_async_copy(z2.at[idxb[p]], rowsb[p], gsem[p]).wait()

    def _process(p):
        rows = rowsb[p]

        def _scale(g, carry2):
            wv = wb[p][pl.ds(16 * g, 16)]
            for j in range(16):
                w = wv[j]
                b = 16 * g + j
                for k in range(8):
                    sl = pl.ds(16 * k, 16)
                    rows[b, sl] = rows[b, sl] * w
            return carry2

        lax.fori_loop(0, _EB // 16, _scale, 0)
        pltpu.sync_copy(rows, acc.at[dstb[p]], add=True)

    row0 = s * _ROWS_PER_SUB
    zoff = c * (_R1 * _NPAD) + _R * _NPAD + row0
    pltpu.sync_copy(z2.at[pl.ds(zoff, _ROWS_PER_SUB)],
                    acc.at[pl.ds(row0, _ROWS_PER_SUB)])
    plsc.subcore_barrier()

    _meta_start(0, 0)
    _meta_wait(0)
    _gather_start(0)
    _meta_start(1, 1)

    def _step(jj, p):
        q = 1 - p

        @pl.when(jj + 1 < _NBLK)
        def _():
            _meta_wait(q)
            _gather_start(q)

        _gather_wait(p)
        _process(p)

        @pl.when(jj + 2 < _NBLK)
        def _():
            _meta_start(jj + 2, p)

    def _blk2(j2, carry):
        _step(2 * j2, 0)
        _step(2 * j2 + 1, 1)
        return carry

    lax.fori_loop(0, _NBLK // 2, _blk2, 0)
    plsc.subcore_barrier()

    pltpu.sync_copy(acc.at[pl.ds(row0, _ROWS_PER_SUB)],
                    out.at[c, pl.ds(row0, _ROWS_PER_SUB)])


def _make_sc_scatter():
    return pl.kernel(
        _sc_scatter_body,
        out_type=jax.ShapeDtypeStruct((_NC, _NPAD, _HALF), jnp.float32),
        mesh=plsc.VectorSubcoreMesh(core_axis_name="c", subcore_axis_name="s"),
        scratch_types=[
            pltpu.VMEM((_EB,), jnp.int32),
            pltpu.VMEM((_EB,), jnp.int32),
            pltpu.VMEM((_EB,), jnp.int32),
            pltpu.VMEM((_EB,), jnp.int32),
            pltpu.VMEM((_EB,), jnp.float32),
            pltpu.VMEM((_EB,), jnp.float32),
            pltpu.VMEM((_EB, _HALF), jnp.float32),
            pltpu.VMEM((_EB, _HALF), jnp.float32),
            pltpu.VMEM_SHARED((_NPAD, _HALF), jnp.float32),
            pltpu.SemaphoreType.DMA,
            pltpu.SemaphoreType.DMA,
            pltpu.SemaphoreType.DMA,
            pltpu.SemaphoreType.DMA,
        ],
    )


_sc_scatter = _make_sc_scatter()


_NT = 2048


def _zmat_body(h_ref, w_ref, z_ref):
    res = jnp.dot(h_ref[...], w_ref[0], preferred_element_type=jnp.float32)
    z_ref[0] = res[:, :_HALF]
    z_ref[1] = res[:, _HALF:]


def _zmat(h, w_all):
    return pl.pallas_call(
        _zmat_body,
        grid=(_NPAD // _NT, _R1),
        in_specs=[
            pl.BlockSpec((_NT, _H), lambda i, r: (i, 0)),
            pl.BlockSpec((1, _H, _H), lambda i, r: (r, 0, 0)),
        ],
        out_specs=pl.BlockSpec((2, _NT, _HALF),
                               lambda i, r: (0, r * (_NPAD // _NT) + i, 0)),
        out_shape=jax.ShapeDtypeStruct((2, _R1 * _NPAD, _HALF), jnp.float32),
    )(h, w_all)


def _norm_body(a0_ref, a1_ref, hin_ref, bc_ref, g_ref, bb_ref, hout_ref):
    conv = jnp.concatenate([a0_ref[...], a1_ref[...]], axis=1)
    t = jnp.maximum(conv + bc_ref[0], 0.0) + hin_ref[...]
    mu = jnp.mean(t, axis=1, keepdims=True)
    var = jnp.mean((t - mu) ** 2, axis=1, keepdims=True)
    hout_ref[...] = (t - mu) * lax.rsqrt(var + 1e-5) * g_ref[0] + bb_ref[0]


def _norm(a0, a1, hin, bc, g, bb):
    nt = 1024
    return pl.pallas_call(
        _norm_body,
        grid=(_NPAD // nt,),
        in_specs=[
            pl.BlockSpec((nt, _HALF), lambda i: (i, 0)),
            pl.BlockSpec((nt, _HALF), lambda i: (i, 0)),
            pl.BlockSpec((nt, _H), lambda i: (i, 0)),
            pl.BlockSpec((1, _H), lambda i: (0, 0)),
            pl.BlockSpec((1, _H), lambda i: (0, 0)),
            pl.BlockSpec((1, _H), lambda i: (0, 0)),
        ],
        out_specs=pl.BlockSpec((nt, _H), lambda i: (i, 0)),
        out_shape=jax.ShapeDtypeStruct((_NPAD, _H), jnp.float32),
    )(a0, a1, hin, bc, g, bb)


def _head_body(h_ref, w1_ref, b1_ref, w2_ref, b2_ref, o_ref):
    u = jnp.dot(h_ref[...], w1_ref[...], preferred_element_type=jnp.float32)
    u = jnp.maximum(u + b1_ref[0], 0.0)
    o_ref[...] = jnp.sum(u * w2_ref[0][None, :], axis=1, keepdims=True) + b2_ref[0, 0]


def _head(h, w1, b1, w2, b2):
    return pl.pallas_call(
        _head_body,
        grid=(_NPAD // _NT,),
        in_specs=[
            pl.BlockSpec((_NT, _H), lambda i: (i, 0)),
            pl.BlockSpec((_H, _DH), lambda i: (0, 0)),
            pl.BlockSpec((1, _DH), lambda i: (0, 0)),
            pl.BlockSpec((1, _DH), lambda i: (0, 0)),
            pl.BlockSpec((1, 1), lambda i: (0, 0)),
        ],
        out_specs=pl.BlockSpec((_NT, 1), lambda i: (i, 0)),
        out_shape=jax.ShapeDtypeStruct((_NPAD, 1), jnp.float32),
    )(h, w1, b1, w2, b2)


def kernel(x, edge_index, edge_attr, edge_type, Wrel, Wroot, bconv, ln_g,
           ln_b, W1, b1, W2, b2):
    src = edge_index[0]
    dst = edge_index[1]
    et = edge_type

    npad_extra = _EP - _E
    all_src = jnp.concatenate([src, jnp.zeros((npad_extra,), jnp.int32)])
    all_dst = jnp.concatenate([dst, jnp.zeros((npad_extra,), jnp.int32)])
    all_et = jnp.concatenate([et, jnp.zeros((npad_extra,), jnp.int32)])
    gid = all_et * _NPAD + all_src
    ridx = jnp.stack([gid, _R1 * _NPAD + gid]).reshape(2, _NS, _NBLK, _EB)
    all_dst = all_dst.reshape(_NS, _NBLK, _EB)

    seg = dst * _R + et
    cnt = jax.ops.segment_sum(jnp.ones((_E,), jnp.float32), seg,
                              num_segments=_N * _R)
    w_real = 1.0 / jnp.clip(cnt, 1.0, None)[seg]

    all_w = jnp.concatenate([w_real, jnp.zeros((npad_extra,), jnp.float32)])
    all_w = all_w.reshape(_NS, _NBLK, _EB)

    h = jnp.pad(x, ((0, _NPAD - _N), (0, 0)))
    w_all = jnp.concatenate([Wrel, Wroot[:, None]], axis=1)

    for l in range(_L):
        z = _zmat(h, w_all[l])
        z2 = z.reshape(2 * _R1 * _NPAD, _HALF)
        acc = _sc_scatter(z2, ridx, all_w, all_dst)
        h = _norm(acc[0], acc[1], h, bconv[l].reshape(1, _H),
                  ln_g[l].reshape(1, _H), ln_b[l].reshape(1, _H))

    out = _head(h, W1, b1.reshape(1, _DH), W2.reshape(1, _DH),
                b2.reshape(1, 1))
    return out[:_N]


# speedup vs baseline: 1.1182x; 1.1182x over previous
import functools

import jax
import jax.numpy as jnp
from jax import lax
from jax.experimental import pallas as pl
from jax.experimental.pallas import tpu as pltpu
from jax.experimental.pallas import tpu_sc as plsc

_N = 10000
_E = 160000
_R = 21
_H = 256
_L = 3
_DH = 128

_NPAD = 10240
_R1 = _R + 1
_HALF = 128
_NC = 2
_NS = 16
_EB = 128
_EV = _E + _N
_EP = 172032
_CHUNK = _EP // _NS
_NBLK = _CHUNK // _EB
_ROWS_PER_SUB = _NPAD // _NS


def _sc_scatter_body(z2, ridx, wvec, dstv, out,
                     idx0, idx1, dst0, dst1, w0, w1,
                     rows0, rows1, acc, g0, g1, m0, m1):
    c = lax.axis_index("c")
    s = lax.axis_index("s")

    idxb = (idx0, idx1)
    dstb = (dst0, dst1)
    wb = (w0, w1)
    rowsb = (rows0, rows1)
    gsem = (g0, g1)
    msem = (m0, m1)

    def _meta_start(jj, p):
        pltpu.async_copy(ridx.at[c, s, jj], idxb[p], msem[p])
        pltpu.async_copy(dstv.at[s, jj], dstb[p], msem[p])
        pltpu.async_copy(wvec.at[s, jj], wb[p], msem[p])

    def _meta_wait(p):
        pltpu.make_async_copy(ridx.at[c, s, 0], idxb[p], msem[p]).wait()
        pltpu.make_async_copy(dstv.at[s, 0], dstb[p], msem[p]).wait()
        pltpu.make_async_copy(wvec.at[s, 0], wb[p], msem[p]).wait()

    def _gather_start(p):
        pltpu.async_copy(z2.at[idxb[p]], rowsb[p], gsem[p])

    def _gather_wait(p):
        pltpu.make_async_copy(z2.at[idxb[p]], rowsb[p], gsem[p]).wait()

    def _process(p):
        rows = rowsb[p]

        def _scale(g, carry2):
            wv = wb[p][pl.ds(16 * g, 16)]
            for j in range(16):
                w = wv[j]
                b = 16 * g + j
                for k in range(8):
                    sl = pl.ds(16 * k, 16)
                    rows[b, sl] = rows[b, sl] * w
            return carry2

        lax.fori_loop(0, _EB // 16, _scale, 0)
        pltpu.sync_copy(rows, acc.at[dstb[p]], add=True)

    zeros16 = jnp.zeros((16,), jnp.float32)

    def _zrow(i, carry):
        for k in range(8):
            rows0[i, pl.ds(16 * k, 16)] = zeros16
        return carry

    lax.fori_loop(0, _EB, _zrow, 0)
    row0 = s * _ROWS_PER_SUB
    for j in range(_ROWS_PER_SUB // _EB):
        pltpu.sync_copy(rows0, acc.at[pl.ds(row0 + j * _EB, _EB)])
    plsc.subcore_barrier()

    _meta_start(0, 0)
    _meta_wait(0)
    _gather_start(0)
    _meta_start(1, 1)

    def _step(jj, p):
        q = 1 - p

        @pl.when(jj + 1 < _NBLK)
        def _():
            _meta_wait(q)
            _gather_start(q)

        _gather_wait(p)
        _process(p)

        @pl.when(jj + 2 < _NBLK)
        def _():
            _meta_start(jj + 2, p)

    def _blk2(j2, carry):
        _step(2 * j2, 0)
        _step(2 * j2 + 1, 1)
        return carry

    lax.fori_loop(0, _NBLK // 2, _blk2, 0)
    plsc.subcore_barrier()

    pltpu.sync_copy(acc.at[pl.ds(row0, _ROWS_PER_SUB)],
                    out.at[c, pl.ds(row0, _ROWS_PER_SUB)])


def _make_sc_scatter():
    return pl.kernel(
        _sc_scatter_body,
        out_type=jax.ShapeDtypeStruct((_NC, _NPAD, _HALF), jnp.float32),
        mesh=plsc.VectorSubcoreMesh(core_axis_name="c", subcore_axis_name="s"),
        scratch_types=[
            pltpu.VMEM((_EB,), jnp.int32),
            pltpu.VMEM((_EB,), jnp.int32),
            pltpu.VMEM((_EB,), jnp.int32),
            pltpu.VMEM((_EB,), jnp.int32),
            pltpu.VMEM((_EB,), jnp.float32),
            pltpu.VMEM((_EB,), jnp.float32),
            pltpu.VMEM((_EB, _HALF), jnp.float32),
            pltpu.VMEM((_EB, _HALF), jnp.float32),
            pltpu.VMEM_SHARED((_NPAD, _HALF), jnp.float32),
            pltpu.SemaphoreType.DMA,
            pltpu.SemaphoreType.DMA,
            pltpu.SemaphoreType.DMA,
            pltpu.SemaphoreType.DMA,
        ],
    )


_sc_scatter = _make_sc_scatter()


_NT = 2048


def _zmat_body(h_ref, w_ref, z_ref):
    res = jnp.dot(h_ref[...].astype(jnp.bfloat16),
                  w_ref[0].astype(jnp.bfloat16),
                  preferred_element_type=jnp.float32)
    z_ref[0] = res[:, :_HALF]
    z_ref[1] = res[:, _HALF:]


def _zmat(h, w_all):
    return pl.pallas_call(
        _zmat_body,
        grid=(_NPAD // _NT, _R1),
        in_specs=[
            pl.BlockSpec((_NT, _H), lambda i, r: (i, 0)),
            pl.BlockSpec((1, _H, _H), lambda i, r: (r, 0, 0)),
        ],
        out_specs=pl.BlockSpec((2, _NT, _HALF),
                               lambda i, r: (0, r * (_NPAD // _NT) + i, 0)),
        out_shape=jax.ShapeDtypeStruct((2, _R1 * _NPAD, _HALF), jnp.float32),
    )(h, w_all)


def _norm_body(a0_ref, a1_ref, hin_ref, bc_ref, g_ref, bb_ref, hout_ref):
    conv = jnp.concatenate([a0_ref[...], a1_ref[...]], axis=1)
    t = jnp.maximum(conv + bc_ref[0], 0.0) + hin_ref[...]
    mu = jnp.mean(t, axis=1, keepdims=True)
    var = jnp.mean((t - mu) ** 2, axis=1, keepdims=True)
    hout_ref[...] = (t - mu) * lax.rsqrt(var + 1e-5) * g_ref[0] + bb_ref[0]


def _norm(a0, a1, hin, bc, g, bb):
    nt = 1024
    return pl.pallas_call(
        _norm_body,
        grid=(_NPAD // nt,),
        in_specs=[
            pl.BlockSpec((nt, _HALF), lambda i: (i, 0)),
            pl.BlockSpec((nt, _HALF), lambda i: (i, 0)),
            pl.BlockSpec((nt, _H), lambda i: (i, 0)),
            pl.BlockSpec((1, _H), lambda i: (0, 0)),
            pl.BlockSpec((1, _H), lambda i: (0, 0)),
            pl.BlockSpec((1, _H), lambda i: (0, 0)),
        ],
        out_specs=pl.BlockSpec((nt, _H), lambda i: (i, 0)),
        out_shape=jax.ShapeDtypeStruct((_NPAD, _H), jnp.float32),
    )(a0, a1, hin, bc, g, bb)


def _head_body(h_ref, w1_ref, b1_ref, w2_ref, b2_ref, o_ref):
    u = jnp.dot(h_ref[...], w1_ref[...], preferred_element_type=jnp.float32)
    u = jnp.maximum(u + b1_ref[0], 0.0)
    o_ref[...] = jnp.sum(u * w2_ref[0][None, :], axis=1, keepdims=True) + b2_ref[0, 0]


def _head(h, w1, b1, w2, b2):
    return pl.pallas_call(
        _head_body,
        grid=(_NPAD // _NT,),
        in_specs=[
            pl.BlockSpec((_NT, _H), lambda i: (i, 0)),
            pl.BlockSpec((_H, _DH), lambda i: (0, 0)),
            pl.BlockSpec((1, _DH), lambda i: (0, 0)),
            pl.BlockSpec((1, _DH), lambda i: (0, 0)),
            pl.BlockSpec((1, 1), lambda i: (0, 0)),
        ],
        out_specs=pl.BlockSpec((_NT, 1), lambda i: (i, 0)),
        out_shape=jax.ShapeDtypeStruct((_NPAD, 1), jnp.float32),
    )(h, w1, b1, w2, b2)


def kernel(x, edge_index, edge_attr, edge_type, Wrel, Wroot, bconv, ln_g,
           ln_b, W1, b1, W2, b2):
    src = edge_index[0]
    dst = edge_index[1]
    et = edge_type

    ar = jnp.arange(_N, dtype=jnp.int32)
    npad_extra = _EP - _EV
    all_src = jnp.concatenate([src, ar, jnp.zeros((npad_extra,), jnp.int32)])
    all_dst = jnp.concatenate([dst, ar, jnp.zeros((npad_extra,), jnp.int32)])
    all_et = jnp.concatenate([et, jnp.full((_N,), _R, jnp.int32),
                              jnp.zeros((npad_extra,), jnp.int32)])
    gid = all_et * _NPAD + all_src
    ridx = jnp.stack([gid, _R1 * _NPAD + gid]).reshape(2, _NS, _NBLK, _EB)
    all_dst = all_dst.reshape(_NS, _NBLK, _EB)

    seg = dst * _R + et
    cnt = jax.ops.segment_sum(jnp.ones((_E,), jnp.float32), seg,
                              num_segments=_N * _R)
    w_real = 1.0 / jnp.clip(cnt, 1.0, None)[seg]

    all_w = jnp.concatenate([w_real, jnp.ones((_N,), jnp.float32),
                             jnp.zeros((npad_extra,), jnp.float32)])
    all_w = all_w.reshape(_NS, _NBLK, _EB)

    h = jnp.pad(x, ((0, _NPAD - _N), (0, 0)))
    w_all = jnp.concatenate([Wrel, Wroot[:, None]], axis=1)

    for l in range(_L):
        z = _zmat(h, w_all[l])
        z2 = z.reshape(2 * _R1 * _NPAD, _HALF)
        acc = _sc_scatter(z2, ridx, all_w, all_dst)
        h = _norm(acc[0], acc[1], h, bconv[l].reshape(1, _H),
                  ln_g[l].reshape(1, _H), ln_b[l].reshape(1, _H))

    out = _head(h, W1, b1.reshape(1, _DH), W2.reshape(1, _DH),
                b2.reshape(1, 1))
    return out[:_N]
